# baseline (speedup 1.0000x reference)
import jax
import jax.numpy as jnp
from jax import lax
from jax.experimental import pallas as pl
from jax.experimental.pallas import tpu as pltpu
from jax.experimental.pallas import tpu_sc as plsc

_LAMBDA = 0.0001
_DIM = 128
_N_EDGES = 320000
_NC = 2
_NS = 16
_NW = _NC * _NS
_LANES = 16
_E_PER_W = _N_EDGES // _NW
_CHUNK = 80
_NCHUNK = _E_PER_W // _CHUNK
_NBUF = 2
_PACKED = _DIM // 2
_GROUPS = _PACKED // _LANES


def _sc_body(x_hbm, ei_hbm, out_hbm,
             ridx_v, cidx_v, rows_v, cols_v, acc_v,
             sem_r0, sem_r1, sem_c0, sem_c1):
    wid = lax.axis_index("s") * _NC + lax.axis_index("c")
    sems_r = (sem_r0, sem_r1)
    sems_c = (sem_c0, sem_c1)

    base = wid * _E_PER_W
    pltpu.sync_copy(ei_hbm.at[pl.ds(base, _E_PER_W)], ridx_v)
    pltpu.sync_copy(ei_hbm.at[pl.ds(_N_EDGES + base, _E_PER_W)], cidx_v)

    def start(g, b):
        pltpu.async_copy(x_hbm.at[ridx_v.at[pl.ds(g * _CHUNK, _CHUNK)]],
                         rows_v.at[b], sems_r[b])
        pltpu.async_copy(x_hbm.at[cidx_v.at[pl.ds(g * _CHUNK, _CHUNK)]],
                         cols_v.at[b], sems_c[b])

    def wait(b):
        pltpu.make_async_copy(x_hbm.at[ridx_v.at[pl.ds(0, _CHUNK)]],
                              rows_v.at[b], sems_r[b]).wait()
        pltpu.make_async_copy(x_hbm.at[cidx_v.at[pl.ds(0, _CHUNK)]],
                              cols_v.at[b], sems_c[b]).wait()

    def compute(b, acc):
        def edge_body(j, accs):
            a0, a1 = accs
            for v in range(_GROUPS):
                r = rows_v[b, j, pl.ds(v * _LANES, _LANES)]
                c = cols_v[b, j, pl.ds(v * _LANES, _LANES)]
                d_hi = (lax.bitcast_convert_type(r, jnp.float32)
                        - lax.bitcast_convert_type(c, jnp.float32))
                d_lo = (lax.bitcast_convert_type(r << 16, jnp.float32)
                        - lax.bitcast_convert_type(c << 16, jnp.float32))
                a0 = a0 + d_hi * d_hi
                a1 = a1 + d_lo * d_lo
            return (a0, a1)
        return lax.fori_loop(0, _CHUNK, edge_body, acc, unroll=4)

    for b in range(_NBUF):
        start(b, b)

    def ring_body(t, acc):
        g = t * _NBUF
        for b in range(_NBUF):
            wait(b)
            acc = compute(b, acc)
            nxt = g + b + _NBUF

            @pl.when(nxt < _NCHUNK)
            def _():
                start(nxt, b)
        return acc

    zero = jnp.zeros((_LANES,), jnp.float32)
    acc = lax.fori_loop(0, _NCHUNK // _NBUF, ring_body, (zero, zero))
    wait(0)
    a0, a1 = compute(0, acc)
    acc_v[...] = a0 + a1
    pltpu.sync_copy(acc_v, out_hbm.at[wid])


@jax.jit
def _sc_loss(xp, ei):
    mesh = plsc.VectorSubcoreMesh(core_axis_name="c", subcore_axis_name="s")
    partials = pl.kernel(
        _sc_body,
        out_type=jax.ShapeDtypeStruct((_NW, _LANES), jnp.float32),
        mesh=mesh,
        compiler_params=pltpu.CompilerParams(use_tc_tiling_on_sc=False),
        scratch_types=[
            pltpu.VMEM((_E_PER_W,), jnp.int32),
            pltpu.VMEM((_E_PER_W,), jnp.int32),
            pltpu.VMEM((_NBUF, _CHUNK, _PACKED), jnp.int32),
            pltpu.VMEM((_NBUF, _CHUNK, _PACKED), jnp.int32),
            pltpu.VMEM((_LANES,), jnp.float32),
            pltpu.SemaphoreType.DMA,
            pltpu.SemaphoreType.DMA,
            pltpu.SemaphoreType.DMA,
            pltpu.SemaphoreType.DMA,
        ],
    )(xp, ei)
    return jnp.sum(partials) * _LAMBDA


def kernel(x, edge_index):
    ei = edge_index.astype(jnp.int32).reshape(2 * _N_EDGES)
    ub = lax.bitcast_convert_type(x.astype(jnp.bfloat16), jnp.uint16)
    lo = ub[:, :_PACKED].astype(jnp.uint32)
    hi = ub[:, _PACKED:].astype(jnp.uint32)
    xp = lax.bitcast_convert_type(lo | (hi << 16), jnp.int32)
    return _sc_loss(xp, ei)


# speedup vs baseline: 1.2012x; 1.2012x over previous
import jax
import jax.numpy as jnp
from jax import lax
from jax.experimental import pallas as pl
from jax.experimental.pallas import tpu as pltpu
from jax.experimental.pallas import tpu_sc as plsc

_LAMBDA = 0.0001
_DIM = 128
_N_EDGES = 320000
_NC = 2
_NS = 16
_NW = _NC * _NS
_LANES = 16
_E_PER_W = _N_EDGES // _NW
_CHUNK = 80
_NCHUNK = _E_PER_W // _CHUNK
_NBUF = 3
_PACKED = _DIM // 2
_GROUPS = _PACKED // _LANES


def _sc_body(x_hbm, ei_hbm, out_hbm,
             ridx_v, cidx_v, rows_v, cols_v, acc_v,
             sem_r0, sem_r1, sem_r2, sem_c0, sem_c1, sem_c2):
    wid = lax.axis_index("s") * _NC + lax.axis_index("c")
    sems_r = (sem_r0, sem_r1, sem_r2)
    sems_c = (sem_c0, sem_c1, sem_c2)

    base = wid * _E_PER_W
    pltpu.sync_copy(ei_hbm.at[pl.ds(base, _E_PER_W)], ridx_v)
    pltpu.sync_copy(ei_hbm.at[pl.ds(_N_EDGES + base, _E_PER_W)], cidx_v)

    def start(g, b):
        pltpu.async_copy(x_hbm.at[ridx_v.at[pl.ds(g * _CHUNK, _CHUNK)]],
                         rows_v.at[b], sems_r[b])
        pltpu.async_copy(x_hbm.at[cidx_v.at[pl.ds(g * _CHUNK, _CHUNK)]],
                         cols_v.at[b], sems_c[b])

    def wait(b):
        pltpu.make_async_copy(x_hbm.at[ridx_v.at[pl.ds(0, _CHUNK)]],
                              rows_v.at[b], sems_r[b]).wait()
        pltpu.make_async_copy(x_hbm.at[cidx_v.at[pl.ds(0, _CHUNK)]],
                              cols_v.at[b], sems_c[b]).wait()

    def compute(b, acc):
        def edge_body(j, accs):
            a0, a1 = accs
            for v in range(_GROUPS):
                r = rows_v[b, j, pl.ds(v * _LANES, _LANES)]
                c = cols_v[b, j, pl.ds(v * _LANES, _LANES)]
                d_hi = (lax.bitcast_convert_type(r, jnp.float32)
                        - lax.bitcast_convert_type(c, jnp.float32))
                d_lo = (lax.bitcast_convert_type(r << 16, jnp.float32)
                        - lax.bitcast_convert_type(c << 16, jnp.float32))
                a0 = a0 + d_hi * d_hi
                a1 = a1 + d_lo * d_lo
            return (a0, a1)
        return lax.fori_loop(0, _CHUNK, edge_body, acc, unroll=4)

    for b in range(_NBUF):
        start(b, b)

    def ring_body(t, acc):
        g = t * _NBUF
        for b in range(_NBUF):
            wait(b)
            acc = compute(b, acc)
            nxt = g + b + _NBUF

            @pl.when(nxt < _NCHUNK)
            def _():
                start(nxt, b)
        return acc

    zero = jnp.zeros((_LANES,), jnp.float32)
    acc = lax.fori_loop(0, _NCHUNK // _NBUF, ring_body, (zero, zero))
    for b in range(_NCHUNK - (_NCHUNK // _NBUF) * _NBUF):
        wait(b)
        acc = compute(b, acc)
    a0, a1 = acc
    acc_v[...] = a0 + a1
    pltpu.sync_copy(acc_v, out_hbm.at[wid])


@jax.jit
def _sc_loss(xp, ei):
    mesh = plsc.VectorSubcoreMesh(core_axis_name="c", subcore_axis_name="s")
    partials = pl.kernel(
        _sc_body,
        out_type=jax.ShapeDtypeStruct((_NW, _LANES), jnp.float32),
        mesh=mesh,
        compiler_params=pltpu.CompilerParams(use_tc_tiling_on_sc=False),
        scratch_types=[
            pltpu.VMEM((_E_PER_W,), jnp.int32),
            pltpu.VMEM((_E_PER_W,), jnp.int32),
            pltpu.VMEM((_NBUF, _CHUNK, _PACKED), jnp.int32),
            pltpu.VMEM((_NBUF, _CHUNK, _PACKED), jnp.int32),
            pltpu.VMEM((_LANES,), jnp.float32),
            pltpu.SemaphoreType.DMA,
            pltpu.SemaphoreType.DMA,
            pltpu.SemaphoreType.DMA,
            pltpu.SemaphoreType.DMA,
            pltpu.SemaphoreType.DMA,
            pltpu.SemaphoreType.DMA,
        ],
    )(xp, ei)
    return jnp.sum(partials) * _LAMBDA


def kernel(x, edge_index):
    ei = edge_index.astype(jnp.int32).reshape(2 * _N_EDGES)
    ub = lax.bitcast_convert_type(x.astype(jnp.bfloat16), jnp.uint16)
    lo = ub[:, :_PACKED].astype(jnp.uint32)
    hi = ub[:, _PACKED:].astype(jnp.uint32)
    xp = lax.bitcast_convert_type(lo | (hi << 16), jnp.int32)
    return _sc_loss(xp, ei)


# speedup vs baseline: 1.2665x; 1.0544x over previous
import jax
import jax.numpy as jnp
from jax import lax
from jax.experimental import pallas as pl
from jax.experimental.pallas import tpu as pltpu
from jax.experimental.pallas import tpu_sc as plsc

_LAMBDA = 0.0001
_DIM = 128
_N_EDGES = 320000
_NC = 2
_NS = 16
_NW = _NC * _NS
_LANES = 16
_E_PER_W = _N_EDGES // _NW
_CHUNK = 80
_NCHUNK = _E_PER_W // _CHUNK
_NBUF = 4
_PACKED = _DIM // 2
_GROUPS = _PACKED // _LANES


def _sc_body(x_hbm, ei_hbm, out_hbm,
             ridx_v, cidx_v, rows_v, cols_v, acc_v,
             sem_r0, sem_r1, sem_r2, sem_r3, sem_c0, sem_c1, sem_c2, sem_c3):
    wid = lax.axis_index("s") * _NC + lax.axis_index("c")
    sems_r = (sem_r0, sem_r1, sem_r2, sem_r3)
    sems_c = (sem_c0, sem_c1, sem_c2, sem_c3)

    base = wid * _E_PER_W
    pltpu.sync_copy(ei_hbm.at[pl.ds(base, _E_PER_W)], ridx_v)
    pltpu.sync_copy(ei_hbm.at[pl.ds(_N_EDGES + base, _E_PER_W)], cidx_v)

    def start(g, b):
        pltpu.async_copy(x_hbm.at[ridx_v.at[pl.ds(g * _CHUNK, _CHUNK)]],
                         rows_v.at[b], sems_r[b])
        pltpu.async_copy(x_hbm.at[cidx_v.at[pl.ds(g * _CHUNK, _CHUNK)]],
                         cols_v.at[b], sems_c[b])

    def wait(b):
        pltpu.make_async_copy(x_hbm.at[ridx_v.at[pl.ds(0, _CHUNK)]],
                              rows_v.at[b], sems_r[b]).wait()
        pltpu.make_async_copy(x_hbm.at[cidx_v.at[pl.ds(0, _CHUNK)]],
                              cols_v.at[b], sems_c[b]).wait()

    def compute(b, acc):
        def edge_body(j, accs):
            a0, a1 = accs
            for v in range(_GROUPS):
                r = rows_v[b, j, pl.ds(v * _LANES, _LANES)]
                c = cols_v[b, j, pl.ds(v * _LANES, _LANES)]
                d_hi = (lax.bitcast_convert_type(r, jnp.float32)
                        - lax.bitcast_convert_type(c, jnp.float32))
                d_lo = (lax.bitcast_convert_type(r << 16, jnp.float32)
                        - lax.bitcast_convert_type(c << 16, jnp.float32))
                a0 = a0 + d_hi * d_hi
                a1 = a1 + d_lo * d_lo
            return (a0, a1)
        return lax.fori_loop(0, _CHUNK, edge_body, acc, unroll=4)

    for b in range(_NBUF):
        start(b, b)

    def ring_body(t, acc):
        g = t * _NBUF
        for b in range(_NBUF):
            wait(b)
            acc = compute(b, acc)
            nxt = g + b + _NBUF

            @pl.when(nxt < _NCHUNK)
            def _():
                start(nxt, b)
        return acc

    zero = jnp.zeros((_LANES,), jnp.float32)
    acc = lax.fori_loop(0, _NCHUNK // _NBUF, ring_body, (zero, zero))
    for b in range(_NCHUNK - (_NCHUNK // _NBUF) * _NBUF):
        wait(b)
        acc = compute(b, acc)
    a0, a1 = acc
    acc_v[...] = a0 + a1
    pltpu.sync_copy(acc_v, out_hbm.at[wid])


@jax.jit
def _sc_loss(xp, ei):
    mesh = plsc.VectorSubcoreMesh(core_axis_name="c", subcore_axis_name="s")
    partials = pl.kernel(
        _sc_body,
        out_type=jax.ShapeDtypeStruct((_NW, _LANES), jnp.float32),
        mesh=mesh,
        compiler_params=pltpu.CompilerParams(use_tc_tiling_on_sc=False),
        scratch_types=[
            pltpu.VMEM((_E_PER_W,), jnp.int32),
            pltpu.VMEM((_E_PER_W,), jnp.int32),
            pltpu.VMEM((_NBUF, _CHUNK, _PACKED), jnp.int32),
            pltpu.VMEM((_NBUF, _CHUNK, _PACKED), jnp.int32),
            pltpu.VMEM((_LANES,), jnp.float32),
            pltpu.SemaphoreType.DMA,
            pltpu.SemaphoreType.DMA,
            pltpu.SemaphoreType.DMA,
            pltpu.SemaphoreType.DMA,
            pltpu.SemaphoreType.DMA,
            pltpu.SemaphoreType.DMA,
            pltpu.SemaphoreType.DMA,
            pltpu.SemaphoreType.DMA,
        ],
    )(xp, ei)
    return jnp.sum(partials) * _LAMBDA


def kernel(x, edge_index):
    ei = edge_index.astype(jnp.int32).reshape(2 * _N_EDGES)
    ub = lax.bitcast_convert_type(x.astype(jnp.bfloat16), jnp.uint16)
    lo = ub[:, :_PACKED].astype(jnp.uint32)
    hi = ub[:, _PACKED:].astype(jnp.uint32)
    xp = lax.bitcast_convert_type(lo | (hi << 16), jnp.int32)
    return _sc_loss(xp, ei)


# speedup vs baseline: 1.2867x; 1.0160x over previous
import jax
import jax.numpy as jnp
from jax import lax
from jax.experimental import pallas as pl
from jax.experimental.pallas import tpu as pltpu
from jax.experimental.pallas import tpu_sc as plsc

_LAMBDA = 0.0001
_DIM = 128
_N_EDGES = 320000
_NC = 2
_NS = 16
_NW = _NC * _NS
_LANES = 16
_E_PER_W = _N_EDGES // _NW
_CHUNK = 80
_NCHUNK = _E_PER_W // _CHUNK
_NBUF = 5
_PACKED = _DIM // 2
_GROUPS = _PACKED // _LANES


def _sc_body(x_hbm, ei_hbm, out_hbm,
             ridx_v, cidx_v, rows_v, cols_v, acc_v,
             sem_r0, sem_r1, sem_r2, sem_r3, sem_r4,
             sem_c0, sem_c1, sem_c2, sem_c3, sem_c4):
    wid = lax.axis_index("s") * _NC + lax.axis_index("c")
    sems_r = (sem_r0, sem_r1, sem_r2, sem_r3, sem_r4)
    sems_c = (sem_c0, sem_c1, sem_c2, sem_c3, sem_c4)

    base = wid * _E_PER_W
    pltpu.sync_copy(ei_hbm.at[pl.ds(base, _E_PER_W)], ridx_v)
    pltpu.sync_copy(ei_hbm.at[pl.ds(_N_EDGES + base, _E_PER_W)], cidx_v)

    def start(g, b):
        pltpu.async_copy(x_hbm.at[ridx_v.at[pl.ds(g * _CHUNK, _CHUNK)]],
                         rows_v.at[b], sems_r[b])
        pltpu.async_copy(x_hbm.at[cidx_v.at[pl.ds(g * _CHUNK, _CHUNK)]],
                         cols_v.at[b], sems_c[b])

    def wait(b):
        pltpu.make_async_copy(x_hbm.at[ridx_v.at[pl.ds(0, _CHUNK)]],
                              rows_v.at[b], sems_r[b]).wait()
        pltpu.make_async_copy(x_hbm.at[cidx_v.at[pl.ds(0, _CHUNK)]],
                              cols_v.at[b], sems_c[b]).wait()

    def compute(b, acc):
        def edge_body(j, accs):
            a0, a1 = accs
            for v in range(_GROUPS):
                r = rows_v[b, j, pl.ds(v * _LANES, _LANES)]
                c = cols_v[b, j, pl.ds(v * _LANES, _LANES)]
                d_hi = (lax.bitcast_convert_type(r, jnp.float32)
                        - lax.bitcast_convert_type(c, jnp.float32))
                d_lo = (lax.bitcast_convert_type(r << 16, jnp.float32)
                        - lax.bitcast_convert_type(c << 16, jnp.float32))
                a0 = a0 + d_hi * d_hi
                a1 = a1 + d_lo * d_lo
            return (a0, a1)
        return lax.fori_loop(0, _CHUNK, edge_body, acc, unroll=4)

    for b in range(_NBUF):
        start(b, b)

    def ring_body(t, acc):
        g = t * _NBUF
        for b in range(_NBUF):
            wait(b)
            acc = compute(b, acc)
            nxt = g + b + _NBUF

            @pl.when(nxt < _NCHUNK)
            def _():
                start(nxt, b)
        return acc

    zero = jnp.zeros((_LANES,), jnp.float32)
    acc = lax.fori_loop(0, _NCHUNK // _NBUF, ring_body, (zero, zero))
    for b in range(_NCHUNK - (_NCHUNK // _NBUF) * _NBUF):
        wait(b)
        acc = compute(b, acc)
    a0, a1 = acc
    acc_v[...] = a0 + a1
    pltpu.sync_copy(acc_v, out_hbm.at[wid])


@jax.jit
def _sc_loss(xp, ei):
    mesh = plsc.VectorSubcoreMesh(core_axis_name="c", subcore_axis_name="s")
    partials = pl.kernel(
        _sc_body,
        out_type=jax.ShapeDtypeStruct((_NW, _LANES), jnp.float32),
        mesh=mesh,
        compiler_params=pltpu.CompilerParams(use_tc_tiling_on_sc=False),
        scratch_types=[
            pltpu.VMEM((_E_PER_W,), jnp.int32),
            pltpu.VMEM((_E_PER_W,), jnp.int32),
            pltpu.VMEM((_NBUF, _CHUNK, _PACKED), jnp.int32),
            pltpu.VMEM((_NBUF, _CHUNK, _PACKED), jnp.int32),
            pltpu.VMEM((_LANES,), jnp.float32),
            pltpu.SemaphoreType.DMA,
            pltpu.SemaphoreType.DMA,
            pltpu.SemaphoreType.DMA,
            pltpu.SemaphoreType.DMA,
            pltpu.SemaphoreType.DMA,
            pltpu.SemaphoreType.DMA,
            pltpu.SemaphoreType.DMA,
            pltpu.SemaphoreType.DMA,
            pltpu.SemaphoreType.DMA,
            pltpu.SemaphoreType.DMA,
        ],
    )(xp, ei)
    return jnp.sum(partials) * _LAMBDA


def kernel(x, edge_index):
    ei = edge_index.astype(jnp.int32).reshape(2 * _N_EDGES)
    ub = lax.bitcast_convert_type(x.astype(jnp.bfloat16), jnp.uint16)
    lo = ub[:, :_PACKED].astype(jnp.uint32)
    hi = ub[:, _PACKED:].astype(jnp.uint32)
    xp = lax.bitcast_convert_type(lo | (hi << 16), jnp.int32)
    return _sc_loss(xp, ei)
